# baseline (speedup 1.0000x reference)
import functools

import numpy as np
import jax
import jax.numpy as jnp
from jax import lax
from jax.experimental import pallas as pl
from jax.experimental.pallas import tpu as pltpu
from jax.experimental.pallas import tpu_sc as plsc

T = 2048
DM = 2048
H = 32
DH = 128
ROPE = 64
TOPK = 512
EPS = 1e-6
HI = lax.Precision.HIGHEST

BT = 128

_NEG = np.float32(-1e9).view(np.int32)
C_NEG = int(_NEG ^ (0x7FFFFFFF & (_NEG >> 31)))
C_BASE = C_NEG - 2048

NC, NS, L = 2, 16, 16
NW = NC * NS
ROWS_PER_W = T // NW


BF = jnp.bfloat16


def _rne16(x):
    i = lax.bitcast_convert_type(x, jnp.int32)
    i2 = i + jnp.int32(0x7FFF) + ((i >> 16) & 1)
    return lax.bitcast_convert_type(i2 & jnp.int32(-65536), jnp.float32)


def _bf(x):
    return _rne16(x).astype(BF)


def _dotbf(a, b, dims):
    return lax.dot_general(_bf(a), _bf(b), dims,
                           preferred_element_type=jnp.float32)


def _prep_kernel(x_ref, wk_ref, wproj_ref, g_ref, b_ref, cos_ref, sin_ref,
                 k_out, w_out):
    x = x_ref[...]
    k = _dotbf(x, wk_ref[...], (((1,), (0,)), ((), ())))
    mu = jnp.mean(k, axis=-1, keepdims=True)
    var = jnp.mean((k - mu) ** 2, axis=-1, keepdims=True)
    k = (k - mu) / jnp.sqrt(var + EPS) * g_ref[...] + b_ref[...]
    kr = k[:, :ROPE]
    k1 = kr[:, :ROPE // 2]
    k2 = kr[:, ROPE // 2:]
    rot = jnp.concatenate([-k2, k1], axis=-1)
    kr = kr * cos_ref[...] + rot * sin_ref[...]
    k_out[...] = jnp.concatenate([kr, k[:, ROPE:]], axis=-1)
    w_out[...] = _dotbf(x, wproj_ref[...], (((1,), (0,)), ((), ()))) * (H ** -0.5)


def _scores_kernel(q_ref, k_ref, w_ref, out_ref):
    pid = pl.program_id(0)
    q = q_ref[...]
    kb = _bf(k_ref[...])
    w = w_ref[...]
    wb = _rne16(w)
    scale = DH ** -0.5
    acc = jnp.zeros((BT, T), jnp.float32)
    for h in range(H):
        qh = q[:, h * DH:(h + 1) * DH]
        logit = lax.dot_general(_bf(qh), kb, (((1,), (1,)), ((), ())),
                                preferred_element_type=jnp.float32)
        r = jnp.maximum(logit * scale, 0.0)
        acc = acc + _rne16(r) * wb[:, h:h + 1]
    row = pid * BT + lax.broadcasted_iota(jnp.int32, (BT, T), 0)
    col = lax.broadcasted_iota(jnp.int32, (BT, T), 1)
    sc = jnp.where(col <= row, acc, jnp.float32(-1e9))
    si = lax.bitcast_convert_type(sc, jnp.int32)
    out_ref[...] = si ^ (jnp.int32(0x7FFFFFFF) & (si >> 31))


def _masked_scores_xs(x, wq_b, wk, w_proj, knorm_g, knorm_b):
    pos = jnp.arange(T, dtype=jnp.float32)
    inv_freq = 1.0 / (10000.0 ** (jnp.arange(0, ROPE, 2, dtype=jnp.float32) / ROPE))
    freqs = pos[:, None] * inv_freq[None, :]
    cos = jnp.concatenate([jnp.cos(freqs), jnp.cos(freqs)], axis=-1)
    sin = jnp.concatenate([jnp.sin(freqs), jnp.sin(freqs)], axis=-1)

    k = x @ wk
    mu = jnp.mean(k, axis=-1, keepdims=True)
    var = jnp.var(k, axis=-1, keepdims=True)
    k = (k - mu) / jnp.sqrt(var + EPS) * knorm_g + knorm_b
    k_rope = k[:, :ROPE]
    rot = jnp.concatenate([-k_rope[:, ROPE // 2:], k_rope[:, :ROPE // 2]],
                          axis=-1)
    k_rope = k_rope * cos + rot * sin
    k_rt = jnp.concatenate([k_rope, k[:, ROPE:]], axis=-1)
    w = (x @ w_proj) * (H ** -0.5)

    q = (x @ wq_b).reshape(T, H, DH)
    q_rope = q[..., :ROPE]
    qrot = jnp.concatenate([-q_rope[..., ROPE // 2:], q_rope[..., :ROPE // 2]],
                           axis=-1)
    q_rope = q_rope * cos[:, None, :] + qrot * sin[:, None, :]
    q_rt = jnp.concatenate([q_rope, q[..., ROPE:]], axis=-1).reshape(T, H * DH)

    grid = (T // BT,)
    xs = pl.pallas_call(
        _scores_kernel,
        grid=grid,
        in_specs=[
            pl.BlockSpec((BT, H * DH), lambda i: (i, 0)),
            pl.BlockSpec((T, DH), lambda i: (0, 0)),
            pl.BlockSpec((BT, H), lambda i: (i, 0)),
        ],
        out_specs=pl.BlockSpec((BT, T), lambda i: (i, 0)),
        out_shape=jax.ShapeDtypeStruct((T, T), jnp.int32),
    )(q_rt, k_rt, w)
    return xs


def _sc_topk_body(xs_hbm, sco_hbm, idx_hbm, ku, idv, so, io):
    wid = lax.axis_index("s") * NC + lax.axis_index("c")
    lane = lax.iota(jnp.int32, L)
    logL = 4

    def cmpex_pair(p, size, dlog):
        d_vr = jnp.int32(1) << (dlog - 4)
        mask = d_vr - 1
        v1 = ((p & ~mask) << 1) | (p & mask)
        o1 = v1 * L
        o2 = o1 + d_vr * L
        ka = ku[pl.ds(o1, L)]
        kb = ku[pl.ds(o2, L)]
        ia = idv[pl.ds(o1, L)]
        ib = idv[pl.ds(o2, L)]
        ev = lane + o1
        descm = (ev & size) == 0
        win = ka > kb
        sel = win == descm
        ku[pl.ds(o1, L)] = jnp.where(sel, ka, kb)
        ku[pl.ds(o2, L)] = jnp.where(sel, kb, ka)
        idv[pl.ds(o1, L)] = jnp.where(sel, ia, ib)
        idv[pl.ds(o2, L)] = jnp.where(sel, ib, ia)

    def vreg_sort(v, size):
        off = v * L
        ev = lane + off
        descm = (ev & size) == 0
        kv = ku[pl.ds(off, L)]
        iv = idv[pl.ds(off, L)]
        kf = jnp.where(descm, kv, ~kv)
        ks, is_ = plsc.sort_key_val(kf, iv, descending=True)
        ku[pl.ds(off, L)] = jnp.where(descm, ks, ~ks)
        idv[pl.ds(off, L)] = is_

    def make_do_row(P):
        logP = P.bit_length() - 1

        def do_row(j, _):
            row = wid + NW * j
            pltpu.sync_copy(xs_hbm.at[row, pl.ds(0, P)], ku.at[pl.ds(0, P)])

            def prep(v, c):
                off = v * L
                ids = lane + off
                idv[pl.ds(off, L)] = ids
                kraw = ku[pl.ds(off, L)]
                ku[pl.ds(off, L)] = jnp.where(kraw == C_NEG,
                                              C_BASE + (2047 - ids), kraw)
                return c
            lax.fori_loop(0, P // L, prep, 0)

            def blockA(v, c):
                vreg_sort(v, jnp.int32(L))
                return c
            lax.fori_loop(0, P // L, blockA, 0)

            def phase(k, c):
                size = jnp.int32(1) << k

                def stage(i, c2):
                    dlog = k - 1 - i

                    def pair(p, c3):
                        cmpex_pair(p, size, dlog)
                        return c3
                    lax.fori_loop(0, P // (2 * L), pair, 0)
                    return c2
                lax.fori_loop(0, k - 4, stage, 0)

                def intra(v, c2):
                    vreg_sort(v, size)
                    return c2
                lax.fori_loop(0, P // L, intra, 0)
                return c
            lax.fori_loop(5, logP + 1, phase, 0)

            def emit(v, c):
                off = v * L
                kf = ku[pl.ds(off, L)]
                sc = lax.bitcast_convert_type(
                    kf ^ (jnp.int32(0x7FFFFFFF) & (kf >> 31)), jnp.float32)
                so[pl.ds(off, L)] = jnp.where(kf < C_NEG, jnp.float32(-1e9), sc)
                io[pl.ds(off, L)] = idv[pl.ds(off, L)]
                return c
            lax.fori_loop(0, TOPK // L, emit, 0)

            pltpu.sync_copy(so, sco_hbm.at[row])
            pltpu.sync_copy(io, idx_hbm.at[row])
            return _

        return do_row

    lax.fori_loop(0, 16, make_do_row(512), 0)
    lax.fori_loop(16, 32, make_do_row(1024), 0)
    lax.fori_loop(32, ROWS_PER_W, make_do_row(2048), 0)


@functools.partial(
    pl.kernel,
    out_type=(jax.ShapeDtypeStruct((T, TOPK), jnp.float32),
              jax.ShapeDtypeStruct((T, TOPK), jnp.int32)),
    mesh=plsc.VectorSubcoreMesh(core_axis_name="c", subcore_axis_name="s",
                                num_cores=NC, num_subcores=NS),
    scratch_types=[
        pltpu.VMEM((T,), jnp.int32),
        pltpu.VMEM((T,), jnp.int32),
        pltpu.VMEM((TOPK,), jnp.float32),
        pltpu.VMEM((TOPK,), jnp.int32),
    ],
    compiler_params=pltpu.CompilerParams(needs_layout_passes=False),
)
def _sc_topk(xs_hbm, sco_hbm, idx_hbm, ku, idv, so, io):
    _sc_topk_body(xs_hbm, sco_hbm, idx_hbm, ku, idv, so, io)


def kernel(x, wq_b, wk, w_proj, knorm_g, knorm_b):
    xs = _masked_scores_xs(x, wq_b, wk, w_proj, knorm_g, knorm_b)
    return _sc_topk(xs)
